# Initial kernel scaffold; baseline (speedup 1.0000x reference)
import jax
import jax.numpy as jnp
from jax.experimental import pallas as pl


def kernel(x, fc1_w, fc1_b, fc1_g, fc1_be, gc_w, gc_b, bn2_g, bn2_be, fc2_w, fc2_b, fc2_g, fc2_be):
    raise NotImplementedError("write your pallas kernel here")


# speedup vs baseline: 10.0824x; 10.0824x over previous
import functools

import jax
import jax.numpy as jnp
from jax.experimental import pallas as pl

K = 9
C_IN = 96
C_HID = 192
BN_EPS = 1e-5
N = 1024
HP = jax.lax.Precision.HIGHEST


def _gelu(x):
    return x * 0.5 * (1.0 + jax.lax.erf(x * 0.7071067811865476))


def _grapher_body(xt_ref, w1_ref, bpre_ref, s1_ref, b1_ref, wa_ref, ba_ref,
                  wb_ref, s2_ref, b2_ref, w3_ref, b3_ref, out_ref):
    X = xt_ref[0]
    F = jax.lax.dot_general(X, w1_ref[...], (((1,), (0,)), ((), ())))
    F = (F + bpre_ref[...][None, :]) * s1_ref[...][None, :] + b1_ref[...][None, :]
    ss = jnp.sum(F * F, axis=1, keepdims=True)
    Xn = F / jnp.sqrt(ss + 1e-12)
    sq = jnp.sum(Xn * Xn, axis=1)
    G = jax.lax.dot_general(Xn, Xn, (((1,), (1,)), ((), ())))
    dist = sq[:, None] - 2.0 * G + sq[None, :]
    A = jax.lax.dot_general(F, wa_ref[...], (((1,), (0,)), ((), ())),
                            precision=HP) + ba_ref[...][None, :]
    Bm = jax.lax.dot_general(F, wb_ref[...], (((1,), (0,)), ((), ())),
                             precision=HP)

    Bmh = Bm.astype(jnp.bfloat16)
    Bml = (Bm - Bmh.astype(jnp.float32)).astype(jnp.bfloat16)

    iota = jax.lax.broadcasted_iota(jnp.int32, (N, N), 1).astype(jnp.float32)
    gmin = jnp.full((N, C_HID), jnp.inf, jnp.float32)
    gmax = jnp.full((N, C_HID), -jnp.inf, jnp.float32)

    def step(_, carry):
        dist, gmin, gmax = carry
        rm = jnp.min(dist, axis=1)
        cand = jnp.where(dist == rm[:, None], iota, 2.0e9)
        idxf = jnp.min(cand, axis=1)
        sel = iota == idxf[:, None]
        onehot = sel.astype(jnp.float32)
        dist = dist + onehot * 1.0e9
        ob = onehot.astype(jnp.bfloat16)
        gath = (jax.lax.dot_general(ob, Bmh, (((1,), (0,)), ((), ())),
                                    preferred_element_type=jnp.float32)
                + jax.lax.dot_general(ob, Bml, (((1,), (0,)), ((), ())),
                                      preferred_element_type=jnp.float32))
        return dist, jnp.minimum(gmin, gath), jnp.maximum(gmax, gath)

    dist, gmin, gmax = jax.lax.fori_loop(0, K, step, (dist, gmin, gmax))

    h = jnp.maximum(_gelu(A + gmin), _gelu(A + gmax))
    h = _gelu(h * s2_ref[...][None, :] + b2_ref[...][None, :])
    out = jax.lax.dot_general(h, w3_ref[...], (((1,), (0,)), ((), ())),
                              precision=HP) + b3_ref[...][None, :]
    out_ref[0] = out + X


@jax.jit
def kernel(x, fc1_w, fc1_b, fc1_g, fc1_be, gc_w, gc_b, bn2_g, bn2_be,
           fc2_w, fc2_b, fc2_g, fc2_be):
    B, C, H, W = x.shape
    n = H * W
    xt = jnp.transpose(x.reshape(B, C, n), (0, 2, 1))

    s1 = fc1_g / jnp.sqrt(1.0 + BN_EPS)
    w1 = fc1_w.T
    w_i = gc_w[:, :C_IN]
    w_j = gc_w[:, C_IN:]
    wa = (w_i - w_j).T
    wb = w_j.T
    s2 = bn2_g / jnp.sqrt(1.0 + BN_EPS)
    s3 = fc2_g / jnp.sqrt(1.0 + BN_EPS)
    w3 = fc2_w.T * s3[None, :]
    b3 = fc2_b * s3 + fc2_be

    grid_spec = pl.GridSpec(
        grid=(B,),
        in_specs=[
            pl.BlockSpec((1, n, C), lambda i: (i, 0, 0)),
            pl.BlockSpec((C, C), lambda i: (0, 0)),
            pl.BlockSpec((C,), lambda i: (0,)),
            pl.BlockSpec((C,), lambda i: (0,)),
            pl.BlockSpec((C,), lambda i: (0,)),
            pl.BlockSpec((C, C_HID), lambda i: (0, 0)),
            pl.BlockSpec((C_HID,), lambda i: (0,)),
            pl.BlockSpec((C, C_HID), lambda i: (0, 0)),
            pl.BlockSpec((C_HID,), lambda i: (0,)),
            pl.BlockSpec((C_HID,), lambda i: (0,)),
            pl.BlockSpec((C_HID, C), lambda i: (0, 0)),
            pl.BlockSpec((C,), lambda i: (0,)),
        ],
        out_specs=pl.BlockSpec((1, n, C), lambda i: (i, 0, 0)),
    )
    out = pl.pallas_call(
        _grapher_body,
        grid_spec=grid_spec,
        out_shape=jax.ShapeDtypeStruct((B, n, C), jnp.float32),
    )(xt, w1, fc1_b, s1, fc1_be, wa, gc_b, wb, s2, bn2_be, w3, b3)
    return jnp.transpose(out, (0, 2, 1)).reshape(B, C, H, W)
